# baseline (speedup 1.0000x reference)
import jax
import jax.numpy as jnp
from jax.experimental import pallas as pl
from jax.experimental.pallas import tpu as pltpu

_P1 = 0.1
_P2 = 1.0
_INF = 1e9
_DW = 9
_D = 81


def _msg(L, m_dw8, m_dw0):
    P = L.shape[1]
    inf8 = jnp.full((8, P), _INF, L.dtype)
    inf1 = jnp.full((1, P), _INF, L.dtype)
    s1 = jnp.concatenate([L[1:], inf1], axis=0)
    s7 = jnp.concatenate([inf1, L[:-1]], axis=0)
    up = jnp.concatenate([s1[8:], inf8], axis=0)
    down = jnp.concatenate([inf8, s7[: _D - 8]], axis=0)
    lf = jnp.where(m_dw8, _INF, s1)
    rt = jnp.where(m_dw0, _INF, s7)
    nmin = jnp.minimum(jnp.minimum(up, down), jnp.minimum(lf, rt))
    minall = jnp.min(L, axis=0, keepdims=True)
    return jnp.minimum(jnp.minimum(L, nmin + _P1), minall + _P2) - minall


def _masks(P):
    d_idx = jax.lax.broadcasted_iota(jnp.int32, (_D, P), 0)
    dw = d_idx % _DW
    return dw == _DW - 1, dw == 0


def _fused_kernel(
    c_ref, eh_ref, ev_ref, o_ref, ch, hout_f, hout_b, vin_f, vin_b, vout_f, vout_b, eh
):
    D, H, W = c_ref.shape
    f32 = c_ref.dtype
    eh[...] = jnp.transpose(eh_ref[...], (0, 2, 1))

    for w0 in range(0, W, 128):
        w1 = min(w0 + 128, W)
        tmp_in = jnp.transpose(c_ref[:, :, w0:w1], (0, 2, 1))
        ch[w0:w1] = jnp.transpose(tmp_in, (1, 0, 2))

    m8h, m0h = _masks(2 * 128)
    zc = jnp.full((D, 128 - H), 0.0, f32)
    zw = jnp.full((128 - H,), 0.0, f32)

    def hstep(k, L2, fb, bb):
        cf = ch[k]
        cb = ch[W - 1 - k]
        c2 = jnp.concatenate([cf, zc, cb, zc], axis=1)
        w2 = jnp.concatenate([eh[0, k], zw, eh[1, W - 1 - k], zw], axis=0)
        L2 = c2 + w2[None, :] * _msg(L2, m8h, m0h)
        hout_f[k - fb] = L2[:, 0:H]
        hout_b[(W - 1 - k) - bb] = L2[:, 128 : 128 + H]
        return L2

    def hflush(buf, wb, w0, first):
        tmp = jnp.transpose(buf[:wb], (1, 0, 2))
        val = jnp.transpose(tmp, (0, 2, 1))
        if first:
            o_ref[:, :, w0 : w0 + wb] = val
        else:
            o_ref[:, :, w0 : w0 + wb] += val

    cf0 = ch[0]
    cb0 = ch[W - 1]
    L2 = jnp.concatenate([cf0, zc, cb0, zc], axis=1)
    hout_f[0] = cf0
    hout_b[55] = cb0

    L2 = jax.lax.fori_loop(1, 56, lambda k, L: hstep(k, L, 0, 256), L2)
    hflush(hout_b, 56, 256, True)
    L2 = jax.lax.fori_loop(56, 128, lambda k, L: hstep(k, L, 0, 128), L2)
    hflush(hout_f, 128, 0, True)
    L2 = jax.lax.fori_loop(128, 184, lambda k, L: hstep(k, L, 128, 128), L2)
    hflush(hout_b, 128, 128, True)
    L2 = jax.lax.fori_loop(184, 256, lambda k, L: hstep(k, L, 128, 0), L2)
    hflush(hout_f, 128, 128, False)
    L2 = jax.lax.fori_loop(256, 312, lambda k, L: hstep(k, L, 256, 0), L2)
    hflush(hout_f, 56, 256, False)
    hflush(hout_b, 128, 0, False)

    WP = 384
    m8v, m0v = _masks(2 * WP)
    zv = jnp.full((D, WP - W), 0.0, f32)
    zwv = jnp.full((WP - W,), 0.0, f32)
    VB = 16

    def vstep(k, L2, s):
        kk = k - VB * s
        cf = vin_f[kk]
        cb = vin_b[VB - 1 - kk]
        c2 = jnp.concatenate([cf, zv, cb, zv], axis=1)
        w2 = jnp.concatenate([ev_ref[0, k], zwv, ev_ref[1, H - 1 - k], zwv], axis=0)
        L2 = c2 + w2[None, :] * _msg(L2, m8v, m0v)
        vout_f[kk] = L2[:, 0:W]
        vout_b[VB - 1 - kk] = L2[:, WP : WP + W]
        return L2

    L2v = None
    for s in range(H // VB):
        f0 = VB * s
        b0 = H - VB - VB * s
        vin_f[...] = jnp.transpose(c_ref[:, f0 : f0 + VB, :], (1, 0, 2))
        vin_b[...] = jnp.transpose(c_ref[:, b0 : b0 + VB, :], (1, 0, 2))
        start = VB * s
        if s == 0:
            cf0v = vin_f[0]
            cb0v = vin_b[VB - 1]
            L2v = jnp.concatenate([cf0v, zv, cb0v, zv], axis=1)
            vout_f[0] = cf0v
            vout_b[VB - 1] = cb0v
            start = 1
        L2v = jax.lax.fori_loop(start, VB * (s + 1), lambda k, L: vstep(k, L, s), L2v)
        o_ref[:, f0 : f0 + VB, :] += jnp.transpose(vout_f[...], (1, 0, 2))
        o_ref[:, b0 : b0 + VB, :] += jnp.transpose(vout_b[...], (1, 0, 2))


def kernel(cost, edge, *, interpret=False):
    c = cost[0]
    D, H, W = c.shape
    f32 = jnp.float32
    out = pl.pallas_call(
        _fused_kernel,
        out_shape=jax.ShapeDtypeStruct((D, H, W), c.dtype),
        scratch_shapes=[
            pltpu.VMEM((W, D, H), f32),
            pltpu.VMEM((128, D, H), f32),
            pltpu.VMEM((128, D, H), f32),
            pltpu.VMEM((16, D, W), f32),
            pltpu.VMEM((16, D, W), f32),
            pltpu.VMEM((16, D, W), f32),
            pltpu.VMEM((16, D, W), f32),
            pltpu.VMEM((2, W, H), f32),
        ],
        interpret=interpret,
    )(c, edge[0, 0:2], edge[0, 2:4])
    return out[None]


# speedup vs baseline: 1.0213x; 1.0213x over previous
import jax
import jax.numpy as jnp
from jax.experimental import pallas as pl
from jax.experimental.pallas import tpu as pltpu

_P1 = 0.1
_P2 = 1.0
_INF = 1e9
_DW = 9
_D = 81


def _msg(L, m_dw8, m_dw0):
    P = L.shape[1]
    inf8 = jnp.full((8, P), _INF, L.dtype)
    inf1 = jnp.full((1, P), _INF, L.dtype)
    s1 = jnp.concatenate([L[1:], inf1], axis=0)
    s7 = jnp.concatenate([inf1, L[:-1]], axis=0)
    up = jnp.concatenate([s1[8:], inf8], axis=0)
    down = jnp.concatenate([inf8, s7[: _D - 8]], axis=0)
    lf = jnp.where(m_dw8, _INF, s1)
    rt = jnp.where(m_dw0, _INF, s7)
    nmin = jnp.minimum(jnp.minimum(up, down), jnp.minimum(lf, rt))
    minall = jnp.min(L, axis=0, keepdims=True)
    return jnp.minimum(jnp.minimum(L, nmin + _P1), minall + _P2) - minall


def _masks(P):
    d_idx = jax.lax.broadcasted_iota(jnp.int32, (_D, P), 0)
    dw = d_idx % _DW
    return dw == _DW - 1, dw == 0


def _fused_kernel(c_ref, eh_ref, ev_ref, o_ref, ch, hout_f, hout_b, vin, vout, eh):
    D, H, W = c_ref.shape
    f32 = c_ref.dtype
    eh[...] = jnp.transpose(eh_ref[...], (0, 2, 1))

    for w0 in range(0, W, 128):
        w1 = min(w0 + 128, W)
        tmp_in = jnp.transpose(c_ref[:, :, w0:w1], (0, 2, 1))
        ch[w0:w1] = jnp.transpose(tmp_in, (1, 0, 2))

    m8h, m0h = _masks(2 * 128)
    zc = jnp.full((D, 128 - H), 0.0, f32)
    zw = jnp.full((128 - H,), 0.0, f32)

    def hstep(k, L2, fb, bb):
        cf = ch[k]
        cb = ch[W - 1 - k]
        c2 = jnp.concatenate([cf, zc, cb, zc], axis=1)
        w2 = jnp.concatenate([eh[0, k], zw, eh[1, W - 1 - k], zw], axis=0)
        L2 = c2 + w2[None, :] * _msg(L2, m8h, m0h)
        hout_f[k - fb] = L2[:, 0:H]
        hout_b[(W - 1 - k) - bb] = L2[:, 128 : 128 + H]
        return L2

    def hflush(buf, wb, w0, first):
        tmp = jnp.transpose(buf[:wb], (1, 0, 2))
        val = jnp.transpose(tmp, (0, 2, 1))
        if first:
            o_ref[:, :, w0 : w0 + wb] = val
        else:
            o_ref[:, :, w0 : w0 + wb] += val

    cf0 = ch[0]
    cb0 = ch[W - 1]
    L2 = jnp.concatenate([cf0, zc, cb0, zc], axis=1)
    hout_f[0] = cf0
    hout_b[55] = cb0

    L2 = jax.lax.fori_loop(1, 56, lambda k, L: hstep(k, L, 0, 256), L2)
    hflush(hout_b, 56, 256, True)
    L2 = jax.lax.fori_loop(56, 128, lambda k, L: hstep(k, L, 0, 128), L2)
    hflush(hout_f, 128, 0, True)
    L2 = jax.lax.fori_loop(128, 184, lambda k, L: hstep(k, L, 128, 128), L2)
    hflush(hout_b, 128, 128, True)
    L2 = jax.lax.fori_loop(184, 256, lambda k, L: hstep(k, L, 128, 0), L2)
    hflush(hout_f, 128, 128, False)
    L2 = jax.lax.fori_loop(256, 312, lambda k, L: hstep(k, L, 256, 0), L2)
    hflush(hout_f, 56, 256, False)
    hflush(hout_b, 128, 0, False)

    m8v, m0v = _masks(W)
    v_blocks = [(h0, min(h0 + 16, H)) for h0 in range(0, H, 16)]

    L = None
    for h0, h1 in v_blocks:
        hb = h1 - h0
        vin[:hb] = jnp.transpose(c_ref[:, h0:h1, :], (1, 0, 2))
        start = 0
        if h0 == 0:
            L = vin[0]
            vout[0] = L
            start = 1

        def vfstep(tt, L, h0=h0):
            L = vin[tt] + ev_ref[0, h0 + tt][None, :] * _msg(L, m8v, m0v)
            vout[tt] = L
            return L

        L = jax.lax.fori_loop(start, hb, vfstep, L)
        o_ref[:, h0:h1, :] += jnp.transpose(vout[:hb], (1, 0, 2))

    for h0, h1 in reversed(v_blocks):
        hb = h1 - h0
        vin[:hb] = jnp.transpose(c_ref[:, h0:h1, :], (1, 0, 2))
        top = hb - 1
        if h1 == H:
            L = vin[hb - 1]
            vout[hb - 1] = L
            top = hb - 2

        def vbstep(i, L, h0=h0, top=top):
            tt = top - i
            L = vin[tt] + ev_ref[1, h0 + tt][None, :] * _msg(L, m8v, m0v)
            vout[tt] = L
            return L

        L = jax.lax.fori_loop(0, top + 1, vbstep, L)
        o_ref[:, h0:h1, :] += jnp.transpose(vout[:hb], (1, 0, 2))


def kernel(cost, edge, *, interpret=False):
    c = cost[0]
    D, H, W = c.shape
    f32 = jnp.float32
    out = pl.pallas_call(
        _fused_kernel,
        out_shape=jax.ShapeDtypeStruct((D, H, W), c.dtype),
        scratch_shapes=[
            pltpu.VMEM((W, D, H), f32),
            pltpu.VMEM((128, D, H), f32),
            pltpu.VMEM((128, D, H), f32),
            pltpu.VMEM((16, D, W), f32),
            pltpu.VMEM((16, D, W), f32),
            pltpu.VMEM((2, W, H), f32),
        ],
        interpret=interpret,
    )(c, edge[0, 0:2], edge[0, 2:4])
    return out[None]
